# Initial kernel scaffold; baseline (speedup 1.0000x reference)
import jax
import jax.numpy as jnp
from jax.experimental import pallas as pl


def kernel(hidden_states, weight):
    raise NotImplementedError("write your pallas kernel here")


# speedup vs baseline: 1.0777x; 1.0777x over previous
import functools

import jax
import jax.numpy as jnp
from jax.experimental import pallas as pl
from jax.experimental.pallas import tpu as pltpu

_TOPK = 8
_NE = 64
_BLK = 512


def _gate_block(x_ref, w_ref, idx_ref, val_ref):
    x = x_ref[...]
    w = w_ref[...]
    logits = jax.lax.dot_general(
        x, w, (((1,), (1,)), ((), ())), preferred_element_type=jnp.float32
    )
    m = jnp.max(logits, axis=-1, keepdims=True)
    e = jnp.exp(logits - m)
    s = jnp.sum(e, axis=-1, keepdims=True)
    lane = jax.lax.broadcasted_iota(jnp.int32, e.shape, 1)
    vals = e
    for k in range(_TOPK):
        mx = jnp.max(vals, axis=-1, keepdims=True)
        idx = jnp.min(jnp.where(vals == mx, lane, _NE), axis=-1, keepdims=True)
        idx_ref[:, k : k + 1] = idx
        val_ref[:, k : k + 1] = mx / s
        vals = jnp.where(lane == idx, -1.0, vals)


@jax.jit
def kernel(hidden_states, weight):
    h = hidden_states.shape[-1]
    x = hidden_states.reshape(-1, h).astype(jnp.float32)
    n = x.shape[0]
    grid = n // _BLK
    idx, val = pl.pallas_call(
        _gate_block,
        grid=(grid,),
        in_specs=[
            pl.BlockSpec((_BLK, h), lambda i: (i, 0)),
            pl.BlockSpec((_NE, h), lambda i: (0, 0)),
        ],
        out_specs=[
            pl.BlockSpec((_BLK, _TOPK), lambda i: (i, 0)),
            pl.BlockSpec((_BLK, _TOPK), lambda i: (i, 0)),
        ],
        out_shape=[
            jax.ShapeDtypeStruct((n, _TOPK), jnp.int32),
            jax.ShapeDtypeStruct((n, _TOPK), jnp.float32),
        ],
    )(x, weight.astype(jnp.float32))
    return idx, val


# speedup vs baseline: 2.0645x; 1.9156x over previous
import functools

import jax
import jax.numpy as jnp
from jax.experimental import pallas as pl
from jax.experimental.pallas import tpu as pltpu

_TOPK = 8
_NE = 64
_BLK = 512


def _gate_block(x_ref, w_ref, idx_ref, val_ref):
    x = x_ref[...]
    w = w_ref[...]
    logits = jax.lax.dot_general(
        w, x, (((1,), (1,)), ((), ())), preferred_element_type=jnp.float32
    )
    m = jnp.max(logits, axis=0, keepdims=True)
    e = jnp.exp(logits - m)
    s = jnp.sum(e, axis=0, keepdims=True)
    row = jax.lax.broadcasted_iota(jnp.int32, e.shape, 0).astype(jnp.float32)
    vals = e
    for k in range(_TOPK):
        mx = jnp.max(vals, axis=0, keepdims=True)
        idx = jnp.min(jnp.where(vals == mx, row, float(_NE)), axis=0, keepdims=True)
        idx_ref[k : k + 1, :] = idx.astype(jnp.int32)
        val_ref[k : k + 1, :] = mx / s
        vals = jnp.where(row == idx, -1.0, vals)


@jax.jit
def kernel(hidden_states, weight):
    h = hidden_states.shape[-1]
    x = hidden_states.reshape(-1, h).astype(jnp.float32)
    n = x.shape[0]
    grid = n // _BLK
    idx_t, val_t = pl.pallas_call(
        _gate_block,
        grid=(grid,),
        in_specs=[
            pl.BlockSpec((_BLK, h), lambda i: (i, 0)),
            pl.BlockSpec((_NE, h), lambda i: (0, 0)),
        ],
        out_specs=[
            pl.BlockSpec((_TOPK, _BLK), lambda i: (0, i)),
            pl.BlockSpec((_TOPK, _BLK), lambda i: (0, i)),
        ],
        out_shape=[
            jax.ShapeDtypeStruct((_TOPK, n), jnp.int32),
            jax.ShapeDtypeStruct((_TOPK, n), jnp.float32),
        ],
    )(x, weight.astype(jnp.float32))
    return idx_t.T, val_t.T


# speedup vs baseline: 2.4758x; 1.1992x over previous
import functools

import jax
import jax.numpy as jnp
from jax.experimental import pallas as pl
from jax.experimental.pallas import tpu as pltpu

_TOPK = 8
_NE = 64
_BLK = 1024


def _gate_block(x_ref, w_ref, idx_ref, val_ref):
    x = x_ref[...]
    w = w_ref[...]
    logits = jax.lax.dot_general(
        w, x, (((1,), (1,)), ((), ())), preferred_element_type=jnp.float32
    )
    m = jnp.max(logits, axis=0, keepdims=True)
    e = jnp.exp(logits - m)
    s = jnp.sum(e, axis=0, keepdims=True)
    row = jax.lax.broadcasted_iota(jnp.int32, e.shape, 0).astype(jnp.float32)
    vals = e
    for k in range(_TOPK):
        mx = jnp.max(vals, axis=0, keepdims=True)
        idx = jnp.min(jnp.where(vals == mx, row, float(_NE)), axis=0, keepdims=True)
        idx_ref[k : k + 1, :] = idx.astype(jnp.int32)
        val_ref[k : k + 1, :] = mx / s
        vals = jnp.where(row == idx, -1.0, vals)


@jax.jit
def kernel(hidden_states, weight):
    h = hidden_states.shape[-1]
    x = hidden_states.reshape(-1, h).astype(jnp.float32)
    n = x.shape[0]
    grid = n // _BLK
    idx_t, val_t = pl.pallas_call(
        _gate_block,
        grid=(grid,),
        in_specs=[
            pl.BlockSpec((_BLK, h), lambda i: (i, 0)),
            pl.BlockSpec((_NE, h), lambda i: (0, 0)),
        ],
        out_specs=[
            pl.BlockSpec((_TOPK, _BLK), lambda i: (0, i)),
            pl.BlockSpec((_TOPK, _BLK), lambda i: (0, i)),
        ],
        out_shape=[
            jax.ShapeDtypeStruct((_TOPK, n), jnp.int32),
            jax.ShapeDtypeStruct((_TOPK, n), jnp.float32),
        ],
    )(x, weight.astype(jnp.float32))
    return idx_t.T, val_t.T


# speedup vs baseline: 2.6262x; 1.0608x over previous
import functools

import jax
import jax.numpy as jnp
from jax.experimental import pallas as pl
from jax.experimental.pallas import tpu as pltpu

_TOPK = 8
_NE = 64
_BLK = 2048


def _gate_block(x_ref, w_ref, idx_ref, val_ref):
    x = x_ref[...]
    w = w_ref[...]
    logits = jax.lax.dot_general(
        w, x, (((1,), (1,)), ((), ())), preferred_element_type=jnp.float32
    )
    m = jnp.max(logits, axis=0, keepdims=True)
    e = jnp.exp(logits - m)
    s = jnp.sum(e, axis=0, keepdims=True)
    row = jax.lax.broadcasted_iota(jnp.int32, e.shape, 0).astype(jnp.float32)
    vals = e
    for k in range(_TOPK):
        mx = jnp.max(vals, axis=0, keepdims=True)
        idx = jnp.min(jnp.where(vals == mx, row, float(_NE)), axis=0, keepdims=True)
        idx_ref[k : k + 1, :] = idx.astype(jnp.int32)
        val_ref[k : k + 1, :] = mx / s
        vals = jnp.where(row == idx, -1.0, vals)


@jax.jit
def kernel(hidden_states, weight):
    h = hidden_states.shape[-1]
    x = hidden_states.reshape(-1, h).astype(jnp.float32)
    n = x.shape[0]
    grid = n // _BLK
    idx_t, val_t = pl.pallas_call(
        _gate_block,
        grid=(grid,),
        in_specs=[
            pl.BlockSpec((_BLK, h), lambda i: (i, 0)),
            pl.BlockSpec((_NE, h), lambda i: (0, 0)),
        ],
        out_specs=[
            pl.BlockSpec((_TOPK, _BLK), lambda i: (0, i)),
            pl.BlockSpec((_TOPK, _BLK), lambda i: (0, i)),
        ],
        out_shape=[
            jax.ShapeDtypeStruct((_TOPK, n), jnp.int32),
            jax.ShapeDtypeStruct((_TOPK, n), jnp.float32),
        ],
    )(x, weight.astype(jnp.float32))
    return idx_t.T, val_t.T
